# baseline (speedup 1.0000x reference)
import functools

import jax
import jax.numpy as jnp
from jax import lax
from jax.experimental import pallas as pl
from jax.experimental.pallas import tpu as pltpu
from jax.experimental.pallas import tpu_sc as plsc

_INPUT_DIM = 3
_PARTITION_NUM = 100
_OUTPUT_DIM = 16
_B = 16384

_info = plsc.get_sparse_core_info()
_NC, _NS, _L = _info.num_cores, _info.num_subcores, _info.num_lanes
_NW = _NC * _NS
_B_PER_W = _B // _NW
_CHUNK = 128
_NCHUNK = _B_PER_W // _CHUNK
_GRP = 128


def _body(inputs_hbm, table_hbm, out_hbm, chunk_v, idx_v, u0, u1, u2, u3,
          g0, g1, g2, g3, rows_v, sem):
    unit_bufs = (u0, u1, u2, u3)
    dst_bufs = (g0, g1, g2, g3)
    wid = lax.axis_index("s") * _NC + lax.axis_index("c")
    base = wid * _B_PER_W

    pltpu.sync_copy(
        inputs_hbm.at[pl.ds(base * _INPUT_DIM, _B_PER_W * _INPUT_DIM)],
        chunk_v)

    lane3 = lax.iota(jnp.int32, _L) * _INPUT_DIM
    copies = []
    for j in range(_NCHUNK):
        for t in range(_CHUNK // _L):
            g = j * _CHUNK + t * _L
            digits = []
            for d in range(_INPUT_DIM):
                x = plsc.load_gather(
                    chunk_v, [lane3 + (g * _INPUT_DIM + d)])
                x = jnp.maximum(x, 0.0)
                s = (x * jnp.float32(_PARTITION_NUM)).astype(jnp.int32)
                digits.append(jnp.minimum(s, _PARTITION_NUM - 1))
            idx = digits[0] + digits[1] * _PARTITION_NUM \
                + digits[2] * (_PARTITION_NUM * _PARTITION_NUM)
            idx_v[pl.ds(g, _L)] = idx
            unit_bufs[j][pl.ds(t * _L, _L)] = idx >> 3
        copies.append(pltpu.async_copy(table_hbm.at[unit_bufs[j]],
                                       dst_bufs[j], sem))

    lane = lax.iota(jnp.int32, _L)
    for j in range(_NCHUNK):
        copies[j].wait()
        for t in range(_CHUNK // _L):
            g = j * _CHUNK + t * _L
            idx = idx_v[pl.ds(g, _L)]
            rem16 = (idx & 7) * _OUTPUT_DIM
            unit_row = t * _L + lane
            for c in range(_OUTPUT_DIM):
                comp = plsc.load_gather(dst_bufs[j], [unit_row, rem16 + c])
                plsc.store_scatter(
                    rows_v, [(g + lane) * _OUTPUT_DIM + c], comp)
    pltpu.sync_copy(rows_v, out_hbm.at[pl.ds(base * _OUTPUT_DIM,
                                             _B_PER_W * _OUTPUT_DIM)])


@jax.jit
def kernel(inputs, table):
    mesh = plsc.VectorSubcoreMesh(core_axis_name="c", subcore_axis_name="s")
    fn = pl.kernel(
        _body,
        mesh=mesh,
        compiler_params=pltpu.CompilerParams(use_tc_tiling_on_sc=True,
                                             needs_layout_passes=False),
        out_type=jax.ShapeDtypeStruct((_B * _OUTPUT_DIM,), jnp.float32),
        scratch_types=[
            pltpu.VMEM((_B_PER_W * _INPUT_DIM,), jnp.float32),
            pltpu.VMEM((_B_PER_W,), jnp.int32),
            pltpu.VMEM((_CHUNK,), jnp.int32),
            pltpu.VMEM((_CHUNK,), jnp.int32),
            pltpu.VMEM((_CHUNK,), jnp.int32),
            pltpu.VMEM((_CHUNK,), jnp.int32),
            pltpu.VMEM((_CHUNK, _GRP), jnp.float32),
            pltpu.VMEM((_CHUNK, _GRP), jnp.float32),
            pltpu.VMEM((_CHUNK, _GRP), jnp.float32),
            pltpu.VMEM((_CHUNK, _GRP), jnp.float32),
            pltpu.VMEM((_B_PER_W * _OUTPUT_DIM,), jnp.float32),
            pltpu.SemaphoreType.DMA,
        ],
    )
    table8 = table[:_PARTITION_NUM ** 3].reshape(-1).reshape(
        _PARTITION_NUM ** 3 // 8, 8 * _OUTPUT_DIM)
    out = fn(inputs.reshape(-1), table8)
    return out.reshape(_B, _OUTPUT_DIM)


# speedup vs baseline: 1.0149x; 1.0149x over previous
import functools

import jax
import jax.numpy as jnp
from jax import lax
from jax.experimental import pallas as pl
from jax.experimental.pallas import tpu as pltpu
from jax.experimental.pallas import tpu_sc as plsc

_INPUT_DIM = 3
_PARTITION_NUM = 100
_OUTPUT_DIM = 16
_B = 16384

_info = plsc.get_sparse_core_info()
_NC, _NS, _L = _info.num_cores, _info.num_subcores, _info.num_lanes
_NW = _NC * _NS
_B_PER_W = _B // _NW
_CHUNK = 128
_NCHUNK = _B_PER_W // _CHUNK


def _body(inputs_hbm, table_hbm, out_hbm, chunk_v, i0, i1, i2, i3,
          r0, r1, r2, r3, sem):
    idx_bufs = (i0, i1, i2, i3)
    row_bufs = (r0, r1, r2, r3)
    wid = lax.axis_index("s") * _NC + lax.axis_index("c")
    base = wid * _B_PER_W

    pltpu.sync_copy(
        inputs_hbm.at[pl.ds(base * _INPUT_DIM, _B_PER_W * _INPUT_DIM)],
        chunk_v)

    lane3 = lax.iota(jnp.int32, _L) * _INPUT_DIM
    copies = []
    for j in range(_NCHUNK):
        for t in range(_CHUNK // _L):
            g = j * _CHUNK + t * _L
            digits = []
            for d in range(_INPUT_DIM):
                x = plsc.load_gather(
                    chunk_v, [lane3 + (g * _INPUT_DIM + d)])
                x = jnp.maximum(x, 0.0)
                s = (x * jnp.float32(_PARTITION_NUM)).astype(jnp.int32)
                digits.append(jnp.minimum(s, _PARTITION_NUM - 1))
            idx = digits[0] + digits[1] * _PARTITION_NUM \
                + digits[2] * (_PARTITION_NUM * _PARTITION_NUM)
            idx_bufs[j][pl.ds(t * _L, _L)] = idx
        copies.append(pltpu.async_copy(table_hbm.at[idx_bufs[j]],
                                       row_bufs[j], sem))

    for j in range(_NCHUNK):
        copies[j].wait()
        pltpu.sync_copy(row_bufs[j],
                        out_hbm.at[pl.ds(base + j * _CHUNK, _CHUNK)])


@jax.jit
def kernel(inputs, table):
    mesh = plsc.VectorSubcoreMesh(core_axis_name="c", subcore_axis_name="s")
    fn = pl.kernel(
        _body,
        mesh=mesh,
        compiler_params=pltpu.CompilerParams(use_tc_tiling_on_sc=False,
                                             needs_layout_passes=False),
        out_type=jax.ShapeDtypeStruct((_B, _OUTPUT_DIM), jnp.float32),
        scratch_types=[
            pltpu.VMEM((_B_PER_W * _INPUT_DIM,), jnp.float32),
            pltpu.VMEM((_CHUNK,), jnp.int32),
            pltpu.VMEM((_CHUNK,), jnp.int32),
            pltpu.VMEM((_CHUNK,), jnp.int32),
            pltpu.VMEM((_CHUNK,), jnp.int32),
            pltpu.VMEM((_CHUNK, _OUTPUT_DIM), jnp.float32),
            pltpu.VMEM((_CHUNK, _OUTPUT_DIM), jnp.float32),
            pltpu.VMEM((_CHUNK, _OUTPUT_DIM), jnp.float32),
            pltpu.VMEM((_CHUNK, _OUTPUT_DIM), jnp.float32),
            pltpu.SemaphoreType.DMA,
        ],
    )
    return fn(inputs.reshape(-1), table)
